# Initial kernel scaffold; baseline (speedup 1.0000x reference)
import jax
import jax.numpy as jnp
from jax.experimental import pallas as pl


def kernel(new_ft, raw_ft, edge_index):
    raise NotImplementedError("write your pallas kernel here")


# speedup vs baseline: 1.7293x; 1.7293x over previous
import functools

import jax
import jax.numpy as jnp
from jax import lax
from jax.experimental import pallas as pl
from jax.experimental.pallas import tpu as pltpu
from jax.experimental.pallas import tpu_sc as plsc

_C = 128
_NW = 32
_L = 16

_DNUMS = lax.GatherDimensionNumbers(
    offset_dims=(), collapsed_slice_dims=(0,), start_index_map=(0,))


def _lane_shuffle(v, perm):
    return lax.gather(v, perm[:, None], _DNUMS, slice_sizes=(1,),
                      mode=lax.GatherScatterMode.PROMISE_IN_BOUNDS)


def _combine(x, y, s, lane):
    m = (lane & s) != 0
    a = jnp.where(m, y, x)
    b = _lane_shuffle(jnp.where(m, x, y), lane ^ s)
    return a + b


def _dot_chunk(urows, vrows, obuf, D):
    lane = lax.iota(jnp.int32, _L)

    def g_body(g, carry):
        row0 = g * _L
        accs = []
        for e in range(_L):
            row = row0 + e
            acc = urows[row, pl.ds(0, _L)] * vrows[row, pl.ds(0, _L)]
            for j in range(1, D // _L):
                acc = acc + (urows[row, pl.ds(j * _L, _L)]
                             * vrows[row, pl.ds(j * _L, _L)])
            accs.append(acc)
        s = 1
        while len(accs) > 1:
            accs = [_combine(accs[2 * i], accs[2 * i + 1], s, lane)
                    for i in range(len(accs) // 2)]
            s *= 2
        obuf[pl.ds(row0, _L)] = accs[0]
        return carry

    lax.fori_loop(0, _C // _L, g_body, 0)


def kernel(new_ft, raw_ft, edge_index):
    N, D = new_ft.shape
    E = edge_index.shape[1]
    assert E % _C == 0
    num_chunks = E // _C

    src = edge_index[0].astype(jnp.int32)
    dst = edge_index[1].astype(jnp.int32)

    mesh = plsc.VectorSubcoreMesh(core_axis_name="c", subcore_axis_name="s")

    @functools.partial(
        pl.kernel,
        mesh=mesh,
        out_type=jax.ShapeDtypeStruct((E,), jnp.float32),
        scratch_types=[
            pltpu.VMEM((_C,), jnp.int32),
            pltpu.VMEM((_C,), jnp.int32),
            pltpu.VMEM((_C, 256), jnp.float32),
            pltpu.VMEM((_C, 256), jnp.float32),
            pltpu.VMEM((_C,), jnp.float32),
            pltpu.SemaphoreType.DMA,
            pltpu.SemaphoreType.DMA,
        ],
    )
    def sc_kernel(new_hbm, raw_hbm, src_hbm, dst_hbm, out_hbm,
                  sidx, didx, urows, vrows, obuf, sem_u, sem_v):
        wid = lax.axis_index("s") * 2 + lax.axis_index("c")

        def chunk_body(t):
            base = t * _C
            pltpu.sync_copy(src_hbm.at[pl.ds(base, _C)], sidx)
            pltpu.sync_copy(dst_hbm.at[pl.ds(base, _C)], didx)
            cu = pltpu.async_copy(new_hbm.at[sidx], urows, sem_u)
            cv = pltpu.async_copy(raw_hbm.at[didx], vrows, sem_v)
            cu.wait()
            cv.wait()
            _dot_chunk(urows, vrows, obuf, D)
            pltpu.sync_copy(obuf, out_hbm.at[pl.ds(base, _C)])

        def w_body(i, carry):
            chunk_body(wid + i * _NW)
            return carry

        lax.fori_loop(0, num_chunks // _NW, w_body, 0)

        rem = num_chunks % _NW
        if rem:
            @pl.when(wid < rem)
            def _():
                chunk_body((num_chunks // _NW) * _NW + wid)

    out = sc_kernel(new_ft, raw_ft, src, dst)
    return out.reshape(E, 1)


# speedup vs baseline: 2.9201x; 1.6886x over previous
import functools

import jax
import jax.numpy as jnp
from jax import lax
from jax.experimental import pallas as pl
from jax.experimental.pallas import tpu as pltpu
from jax.experimental.pallas import tpu_sc as plsc

_C = 128
_NW = 32
_L = 16

_DNUMS = lax.GatherDimensionNumbers(
    offset_dims=(), collapsed_slice_dims=(0,), start_index_map=(0,))


def _lane_shuffle(v, perm):
    return lax.gather(v, perm[:, None], _DNUMS, slice_sizes=(1,),
                      mode=lax.GatherScatterMode.PROMISE_IN_BOUNDS)


def _combine(x, y, s, lane):
    m = (lane & s) != 0
    a = jnp.where(m, y, x)
    b = _lane_shuffle(jnp.where(m, x, y), lane ^ s)
    return a + b


def _dot_chunk(urows, vrows, obuf, D):
    lane = lax.iota(jnp.int32, _L)

    def g_body(g, carry):
        row0 = g * _L
        def e_body(e, tot):
            row = row0 + 2 * e
            for k in range(2):
                r = row + k
                acc0 = urows[r, pl.ds(0, _L)] * vrows[r, pl.ds(0, _L)]
                acc1 = urows[r, pl.ds(_L, _L)] * vrows[r, pl.ds(_L, _L)]
                for j in range(2, D // _L, 2):
                    acc0 = acc0 + (urows[r, pl.ds(j * _L, _L)]
                                   * vrows[r, pl.ds(j * _L, _L)])
                    acc1 = acc1 + (urows[r, pl.ds((j + 1) * _L, _L)]
                                   * vrows[r, pl.ds((j + 1) * _L, _L)])
                acc = acc0 + acc1
                for s in (1, 2, 4, 8):
                    acc = acc + _lane_shuffle(acc, lane ^ s)
                tot = jnp.where(lane == 2 * e + k, acc, tot)
            return tot

        tot = lax.fori_loop(0, _L // 2, e_body, jnp.zeros((_L,), jnp.float32))
        obuf[pl.ds(row0, _L)] = tot
        return carry

    lax.fori_loop(0, _C // _L, g_body, 0)


def kernel(new_ft, raw_ft, edge_index):
    N, D = new_ft.shape
    E = edge_index.shape[1]
    assert E % _C == 0
    num_chunks = E // _C

    src = edge_index[0].astype(jnp.int32)
    dst = edge_index[1].astype(jnp.int32)

    mesh = plsc.VectorSubcoreMesh(core_axis_name="c", subcore_axis_name="s")

    @functools.partial(
        pl.kernel,
        mesh=mesh,
        out_type=jax.ShapeDtypeStruct((E,), jnp.float32),
        scratch_types=[
            pltpu.VMEM((_C,), jnp.int32),
            pltpu.VMEM((_C,), jnp.int32),
            pltpu.VMEM((_C, 256), jnp.float32),
            pltpu.VMEM((_C, 256), jnp.float32),
            pltpu.VMEM((_C,), jnp.float32),
            pltpu.SemaphoreType.DMA,
            pltpu.SemaphoreType.DMA,
        ],
    )
    def sc_kernel(new_hbm, raw_hbm, src_hbm, dst_hbm, out_hbm,
                  sidx, didx, urows, vrows, obuf, sem_u, sem_v):
        wid = lax.axis_index("s") * 2 + lax.axis_index("c")

        def chunk_body(t):
            base = t * _C
            pltpu.sync_copy(src_hbm.at[pl.ds(base, _C)], sidx)
            pltpu.sync_copy(dst_hbm.at[pl.ds(base, _C)], didx)
            cu = pltpu.async_copy(new_hbm.at[sidx], urows, sem_u)
            cv = pltpu.async_copy(raw_hbm.at[didx], vrows, sem_v)
            cu.wait()
            cv.wait()
            _dot_chunk(urows, vrows, obuf, D)
            pltpu.sync_copy(obuf, out_hbm.at[pl.ds(base, _C)])

        def w_body(i, carry):
            chunk_body(wid + i * _NW)
            return carry

        lax.fori_loop(0, num_chunks // _NW, w_body, 0)

        rem = num_chunks % _NW
        if rem:
            @pl.when(wid < rem)
            def _():
                chunk_body((num_chunks // _NW) * _NW + wid)

    out = sc_kernel(new_ft, raw_ft, src, dst)
    return out.reshape(E, 1)


# speedup vs baseline: 4.1643x; 1.4261x over previous
import functools

import jax
import jax.numpy as jnp
from jax import lax
from jax.experimental import pallas as pl
from jax.experimental.pallas import tpu as pltpu
from jax.experimental.pallas import tpu_sc as plsc

_C = 64
_NW = 32
_L = 16

_DNUMS = lax.GatherDimensionNumbers(
    offset_dims=(), collapsed_slice_dims=(0,), start_index_map=(0,))


def _lane_shuffle(v, perm):
    return lax.gather(v, perm[:, None], _DNUMS, slice_sizes=(1,),
                      mode=lax.GatherScatterMode.PROMISE_IN_BOUNDS)


def _dot_chunk(urows, vrows, obuf, D):
    lane = lax.iota(jnp.int32, _L)

    def g_body(g, carry):
        row0 = g * _L

        def e_body(e, tot):
            row = row0 + 2 * e
            for k in range(2):
                r = row + k
                acc0 = urows[r, pl.ds(0, _L)] * vrows[r, pl.ds(0, _L)]
                acc1 = urows[r, pl.ds(_L, _L)] * vrows[r, pl.ds(_L, _L)]
                for j in range(2, D // _L, 2):
                    acc0 = acc0 + (urows[r, pl.ds(j * _L, _L)]
                                   * vrows[r, pl.ds(j * _L, _L)])
                    acc1 = acc1 + (urows[r, pl.ds((j + 1) * _L, _L)]
                                   * vrows[r, pl.ds((j + 1) * _L, _L)])
                acc = acc0 + acc1
                for s in (1, 2, 4, 8):
                    acc = acc + _lane_shuffle(acc, lane ^ s)
                tot = jnp.where(lane == 2 * e + k, acc, tot)
            return tot

        tot = lax.fori_loop(0, _L // 2, e_body, jnp.zeros((_L,), jnp.float32))
        obuf[pl.ds(row0, _L)] = tot
        return carry

    lax.fori_loop(0, _C // _L, g_body, 0)


def kernel(new_ft, raw_ft, edge_index):
    N, D = new_ft.shape
    E = edge_index.shape[1]
    assert E % _C == 0
    num_chunks = E // _C
    nfull = num_chunks // _NW
    rem = num_chunks % _NW

    src = edge_index[0].astype(jnp.int32)
    dst = edge_index[1].astype(jnp.int32)

    mesh = plsc.VectorSubcoreMesh(core_axis_name="c", subcore_axis_name="s")

    @functools.partial(
        pl.kernel,
        mesh=mesh,
        out_type=jax.ShapeDtypeStruct((E,), jnp.float32),
        scratch_types=[
            pltpu.VMEM((_C,), jnp.int32),
            pltpu.VMEM((_C,), jnp.int32),
            pltpu.VMEM((_C,), jnp.int32),
            pltpu.VMEM((_C,), jnp.int32),
            pltpu.VMEM((_C, 256), jnp.float32),
            pltpu.VMEM((_C, 256), jnp.float32),
            pltpu.VMEM((_C, 256), jnp.float32),
            pltpu.VMEM((_C, 256), jnp.float32),
            pltpu.VMEM((_C,), jnp.float32),
            pltpu.SemaphoreType.DMA,
            pltpu.SemaphoreType.DMA,
            pltpu.SemaphoreType.DMA,
            pltpu.SemaphoreType.DMA,
        ],
    )
    def sc_kernel(new_hbm, raw_hbm, src_hbm, dst_hbm, out_hbm,
                  sidx0, didx0, sidx1, didx1,
                  urows0, vrows0, urows1, vrows1, obuf,
                  su0, sv0, su1, sv1):
        wid = lax.axis_index("s") * 2 + lax.axis_index("c")
        n_me = jnp.where(wid < rem, nfull + 1, nfull) if rem else nfull

        bufs = ((sidx0, didx0, urows0, vrows0, su0, sv0),
                (sidx1, didx1, urows1, vrows1, su1, sv1))

        def start_gathers(i):
            base = (wid + i * _NW) * _C

            def go(sidx, didx, ub, vb, su, sv):
                pltpu.sync_copy(src_hbm.at[pl.ds(base, _C)], sidx)
                pltpu.sync_copy(dst_hbm.at[pl.ds(base, _C)], didx)
                pltpu.make_async_copy(new_hbm.at[sidx], ub, su).start()
                pltpu.make_async_copy(raw_hbm.at[didx], vb, sv).start()

            for b in range(2):
                @pl.when(i % 2 == b)
                def _(b=b):
                    go(*bufs[b])

        def body(i, carry):
            @pl.when(i + 1 < n_me)
            def _():
                start_gathers(i + 1)

            base = (wid + i * _NW) * _C
            for b in range(2):
                @pl.when(i % 2 == b)
                def _(b=b):
                    sidx, didx, ub, vb, su, sv = bufs[b]
                    pltpu.make_async_copy(new_hbm.at[sidx], ub, su).wait()
                    pltpu.make_async_copy(raw_hbm.at[didx], vb, sv).wait()
                    _dot_chunk(ub, vb, obuf, D)
            pltpu.sync_copy(obuf, out_hbm.at[pl.ds(base, _C)])
            return carry

        start_gathers(0)
        lax.fori_loop(0, n_me, body, 0)

    out = sc_kernel(new_ft, raw_ft, src, dst)
    return out.reshape(E, 1)
